# baseline (speedup 1.0000x reference)
import functools

import jax
import jax.numpy as jnp
from jax import lax
from jax.experimental import pallas as pl
from jax.experimental.pallas import tpu as pltpu
from jax.experimental.pallas import tpu_sc as plsc

N_STATES = 1000000
H = 64
B = 16384

_SBLK = 65536
_NBLK = (N_STATES + _SBLK - 1) // _SBLK
_SPAD = _NBLK * _SBLK
_NTILE = _SPAD // 128

_info = plsc.get_sparse_core_info()
_NC, _NS = _info.num_cores, _info.num_subcores
_NW = _NC * _NS
_BPW = B // _NW
_CH = 128
_NCH = _BPW // _CH

_mesh = plsc.VectorSubcoreMesh(core_axis_name="c", subcore_axis_name="s")


def _tc_sweep_body(tablet_ref, w_ref, b_ref, y_ref):
    h = jnp.tanh(tablet_ref[...])
    y = jax.lax.dot_general(
        w_ref[...], h, (((1,), (0,)), ((), ())),
        preferred_element_type=jnp.float32,
    )
    y_ref[...] = y + b_ref[...]


@functools.partial(
    pl.kernel,
    mesh=_mesh,
    out_type=jax.ShapeDtypeStruct((B, 8), jnp.float32),
    scratch_types=[
        pltpu.VMEM((_NCH, _CH), jnp.int32),
        pltpu.VMEM((_CH,), jnp.int32),
        pltpu.VMEM((_CH, 2, 128), jnp.float32),
        pltpu.VMEM((_BPW, 8), jnp.float32),
        pltpu.SemaphoreType.DMA,
    ],
    compiler_params=pltpu.CompilerParams(
        use_tc_tiling_on_sc=False, needs_layout_passes=False
    ),
)
def _sc_gather(idx_hbm, y3_hbm, out_hbm, idx_v, tidx_v, tiles_v, rows_v, sem):
    wid = lax.axis_index("s") * _NC + lax.axis_index("c")
    base = wid * _BPW
    pltpu.sync_copy(idx_hbm.at[wid], idx_v)
    zeros16 = jnp.zeros((16,), jnp.int32)
    ones16 = zeros16 + 1
    iota16 = lax.iota(jnp.int32, 16)
    for j in range(_NCH):
        for g in range(_CH // 16):
            vec = idx_v[j, pl.ds(g * 16, 16)]
            tidx_v[pl.ds(g * 16, 16)] = vec >> 7
        pltpu.async_copy(y3_hbm.at[tidx_v], tiles_v, sem).wait()
        for g in range(_CH // 16):
            vec = idx_v[j, pl.ds(g * 16, 16)]
            lane = vec & 127
            slot = iota16 + (g * 16)
            y0 = plsc.load_gather(tiles_v, [slot, zeros16, lane])
            y1 = plsc.load_gather(tiles_v, [slot, ones16, lane])
            rslot = slot + (j * _CH)
            plsc.store_scatter(rows_v, [rslot, zeros16], y0)
            plsc.store_scatter(rows_v, [rslot, ones16], y1)
    pltpu.sync_copy(rows_v, out_hbm.at[pl.ds(base, _BPW)])


def kernel(state_index, emb_table, lin_w, lin_b):
    idx = state_index.astype(jnp.int32).reshape(_NW, _NCH, _CH)
    tablet = emb_table.T
    y = pl.pallas_call(
        _tc_sweep_body,
        grid=(_NBLK,),
        in_specs=[
            pl.BlockSpec((H, _SBLK), lambda k: (0, k)),
            pl.BlockSpec((2, H), lambda k: (0, 0)),
            pl.BlockSpec((2, 1), lambda k: (0, 0)),
        ],
        out_specs=pl.BlockSpec((2, _SBLK), lambda k: (0, k)),
        out_shape=jax.ShapeDtypeStruct((2, _SPAD), jnp.float32),
    )(tablet, lin_w, lin_b.reshape(2, 1))
    y3 = y.reshape(2, _NTILE, 128).transpose(1, 0, 2)
    return _sc_gather(idx, y3)[:, :2]


# speedup vs baseline: 1.0220x; 1.0220x over previous
import functools

import jax
import jax.numpy as jnp
from jax import lax
from jax.experimental import pallas as pl
from jax.experimental.pallas import tpu as pltpu
from jax.experimental.pallas import tpu_sc as plsc

N_STATES = 1000000
H = 64
B = 16384

_SBLK = 32768
_NBLK = (N_STATES + _SBLK - 1) // _SBLK
_SPAD = _NBLK * _SBLK
_NTILE = _SPAD // 128

_info = plsc.get_sparse_core_info()
_NC, _NS = _info.num_cores, _info.num_subcores
_NW = _NC * _NS
_BPW = B // _NW
_CH = 128
_NCH = _BPW // _CH

_mesh = plsc.VectorSubcoreMesh(core_axis_name="c", subcore_axis_name="s")


def _tc_sweep_body(tablet_ref, w_ref, b_ref, y_ref):
    h = jnp.tanh(tablet_ref[...])
    y = jax.lax.dot_general(
        w_ref[...], h, (((1,), (0,)), ((), ())),
        preferred_element_type=jnp.float32,
    )
    y_ref[...] = y + b_ref[...]


@functools.partial(
    pl.kernel,
    mesh=_mesh,
    out_type=jax.ShapeDtypeStruct((B, 8), jnp.float32),
    scratch_types=[
        pltpu.VMEM((_NCH, _CH), jnp.int32),
        pltpu.VMEM((_CH,), jnp.int32),
        pltpu.VMEM((_CH, 2, 128), jnp.float32),
        pltpu.VMEM((_BPW, 8), jnp.float32),
        pltpu.SemaphoreType.DMA,
    ],
    compiler_params=pltpu.CompilerParams(
        use_tc_tiling_on_sc=False, needs_layout_passes=False
    ),
)
def _sc_gather(idx_hbm, y3_hbm, out_hbm, idx_v, tidx_v, tiles_v, rows_v, sem):
    wid = lax.axis_index("s") * _NC + lax.axis_index("c")
    base = wid * _BPW
    pltpu.sync_copy(idx_hbm.at[wid], idx_v)
    zeros16 = jnp.zeros((16,), jnp.int32)
    ones16 = zeros16 + 1
    iota16 = lax.iota(jnp.int32, 16)
    for j in range(_NCH):
        for g in range(_CH // 16):
            vec = idx_v[j, pl.ds(g * 16, 16)]
            tidx_v[pl.ds(g * 16, 16)] = vec >> 7
        pltpu.async_copy(y3_hbm.at[tidx_v], tiles_v, sem).wait()
        for g in range(_CH // 16):
            vec = idx_v[j, pl.ds(g * 16, 16)]
            lane = vec & 127
            slot = iota16 + (g * 16)
            y0 = plsc.load_gather(tiles_v, [slot, zeros16, lane])
            y1 = plsc.load_gather(tiles_v, [slot, ones16, lane])
            rslot = slot + (j * _CH)
            plsc.store_scatter(rows_v, [rslot, zeros16], y0)
            plsc.store_scatter(rows_v, [rslot, ones16], y1)
    pltpu.sync_copy(rows_v, out_hbm.at[pl.ds(base, _BPW)])


def kernel(state_index, emb_table, lin_w, lin_b):
    idx = state_index.astype(jnp.int32).reshape(_NW, _NCH, _CH)
    tablet = emb_table.T
    y = pl.pallas_call(
        _tc_sweep_body,
        grid=(_NBLK,),
        in_specs=[
            pl.BlockSpec((H, _SBLK), lambda k: (0, k)),
            pl.BlockSpec((2, H), lambda k: (0, 0)),
            pl.BlockSpec((2, 1), lambda k: (0, 0)),
        ],
        out_specs=pl.BlockSpec((2, _SBLK), lambda k: (0, k)),
        out_shape=jax.ShapeDtypeStruct((2, _SPAD), jnp.float32),
    )(tablet, lin_w, lin_b.reshape(2, 1))
    y3 = y.reshape(2, _NTILE, 128).transpose(1, 0, 2)
    return _sc_gather(idx, y3)[:, :2]


# speedup vs baseline: 1.1664x; 1.1413x over previous
import functools

import jax
import jax.numpy as jnp
from jax import lax
from jax.experimental import pallas as pl
from jax.experimental.pallas import tpu as pltpu
from jax.experimental.pallas import tpu_sc as plsc

N_STATES = 1000000
H = 64
B = 16384

_SBLK = 32768
_NBLK = (N_STATES + _SBLK - 1) // _SBLK
_SPAD = _NBLK * _SBLK
_NTILE = _SPAD // 128

_info = plsc.get_sparse_core_info()
_NC, _NS = _info.num_cores, _info.num_subcores
_NW = _NC * _NS
_BPW = B // _NW
_CH = 128
_NCH = _BPW // _CH

_mesh = plsc.VectorSubcoreMesh(core_axis_name="c", subcore_axis_name="s")


def _tc_sweep_body(tablet_ref, w_ref, b_ref, y_ref):
    h = jnp.tanh(tablet_ref[...])
    y = jax.lax.dot_general(
        w_ref[...], h, (((1,), (0,)), ((), ())),
        preferred_element_type=jnp.float32,
    )
    y_ref[...] = y + b_ref[...]


@functools.partial(
    pl.kernel,
    mesh=_mesh,
    out_type=jax.ShapeDtypeStruct((B // 128, 2, 128), jnp.float32),
    scratch_types=[
        pltpu.VMEM((_NCH, _CH), jnp.int32),
        pltpu.VMEM((_NCH, _CH), jnp.int32),
        pltpu.VMEM((2, _CH, 2, 128), jnp.float32),
        pltpu.VMEM((_NCH, 2, _CH), jnp.float32),
        pltpu.SemaphoreType.DMA,
        pltpu.SemaphoreType.DMA,
    ],
    compiler_params=pltpu.CompilerParams(
        use_tc_tiling_on_sc=False, needs_layout_passes=False
    ),
)
def _sc_gather(idx_hbm, y3_hbm, out_hbm, idx_v, tidx_v, tiles_v, rows_v, s0, s1):
    wid = lax.axis_index("s") * _NC + lax.axis_index("c")
    pltpu.sync_copy(idx_hbm.at[wid], idx_v)
    zeros16 = jnp.zeros((16,), jnp.int32)
    ones16 = zeros16 + 1
    iota16 = lax.iota(jnp.int32, 16)
    sems = [s0, s1]
    for j in range(_NCH):
        for g in range(_CH // 16):
            vec = idx_v[j, pl.ds(g * 16, 16)]
            tidx_v[j, pl.ds(g * 16, 16)] = vec >> 7
    fires = []
    for j in range(min(2, _NCH)):
        fires.append(
            pltpu.async_copy(
                y3_hbm.at[tidx_v.at[j]], tiles_v.at[j % 2], sems[j % 2]
            )
        )
    for j in range(_NCH):
        fires[j].wait()
        buf = tiles_v.at[j % 2]
        for g in range(_CH // 16):
            vec = idx_v[j, pl.ds(g * 16, 16)]
            lane = vec & 127
            slot = iota16 + (g * 16)
            y0 = plsc.load_gather(buf, [slot, zeros16, lane])
            y1 = plsc.load_gather(buf, [slot, ones16, lane])
            rows_v[j, 0, pl.ds(g * 16, 16)] = y0
            rows_v[j, 1, pl.ds(g * 16, 16)] = y1
        if j + 2 < _NCH:
            fires.append(
                pltpu.async_copy(
                    y3_hbm.at[tidx_v.at[j + 2]], tiles_v.at[j % 2], sems[j % 2]
                )
            )
    pltpu.sync_copy(rows_v, out_hbm.at[pl.ds(wid * _NCH, _NCH)])


def kernel(state_index, emb_table, lin_w, lin_b):
    idx = state_index.astype(jnp.int32).reshape(_NW, _NCH, _CH)
    tablet = emb_table.T
    y = pl.pallas_call(
        _tc_sweep_body,
        grid=(_NBLK,),
        in_specs=[
            pl.BlockSpec((H, _SBLK), lambda k: (0, k)),
            pl.BlockSpec((2, H), lambda k: (0, 0)),
            pl.BlockSpec((2, 1), lambda k: (0, 0)),
        ],
        out_specs=pl.BlockSpec((2, _SBLK), lambda k: (0, k)),
        out_shape=jax.ShapeDtypeStruct((2, _SPAD), jnp.float32),
    )(tablet, lin_w, lin_b.reshape(2, 1))
    y3 = y.reshape(2, _NTILE, 128).transpose(1, 0, 2)
    out3 = _sc_gather(idx, y3)
    return out3.transpose(1, 0, 2).reshape(2, B).T
